# Initial kernel scaffold; baseline (speedup 1.0000x reference)
import jax
import jax.numpy as jnp
from jax.experimental import pallas as pl


def kernel(merged_tree_feature, merged_tree_edge_index, indices, emb_table, h0, W_ih0, W_hh0, b_ih0, b_hh0, W_ih1, W_hh1, b_ih1, b_hh1, W1, b1, W2, b2):
    raise NotImplementedError("write your pallas kernel here")


# speedup vs baseline: 1.2388x; 1.2388x over previous
import functools

import jax
import jax.numpy as jnp
from jax.experimental import pallas as pl
from jax.experimental.pallas import tpu as pltpu

N = 10000
L = 20
V = 30000
DIN = 100
H = 128
H1 = 100
H2 = 100
E = 160000
B = 128

NB = 1000
NBLK = N // NB


def _gru_body(x_ref, h00_ref, h01_ref,
              wih0_ref, whh0_ref, wih1_ref, whh1_ref,
              bih0_ref, bhh0_ref, bih1_ref, bhh1_ref,
              out_ref, ha, hb):
    t = pl.program_id(1)

    @pl.when(t == 0)
    def _init():
        ha[...] = h00_ref[...]
        hb[...] = h01_ref[...]

    def gates(gi, gh, h_prev):
        i_r, i_z, i_n = gi[:, :H], gi[:, H:2 * H], gi[:, 2 * H:]
        h_r, h_z, h_n = gh[:, :H], gh[:, H:2 * H], gh[:, 2 * H:]
        r = jax.nn.sigmoid(i_r + h_r)
        z = jax.nn.sigmoid(i_z + h_z)
        n = jnp.tanh(i_n + r * h_n)
        return (1.0 - z) * n + z * h_prev

    x_t = x_ref[:, 0, 0, :]
    h_a = ha[...]
    h_b = hb[...]
    gi0 = jnp.dot(x_t, wih0_ref[...], preferred_element_type=jnp.float32) + bih0_ref[...]
    gh0 = jnp.dot(h_a, whh0_ref[...], preferred_element_type=jnp.float32) + bhh0_ref[...]
    h_a = gates(gi0, gh0, h_a)
    gi1 = jnp.dot(h_a, wih1_ref[...], preferred_element_type=jnp.float32) + bih1_ref[...]
    gh1 = jnp.dot(h_b, whh1_ref[...], preferred_element_type=jnp.float32) + bhh1_ref[...]
    h_b = gates(gi1, gh1, h_b)
    ha[...] = h_a
    hb[...] = h_b

    @pl.when(t == L - 1)
    def _fin():
        out_ref[...] = h_b


def _gru(x_all, h00, h01, wih0t, whh0t, wih1t, whh1t, bih0, bhh0, bih1, bhh1):
    grid = (NBLK, L)
    full = lambda shape: pl.BlockSpec(shape, lambda n, t: (0,) * len(shape))
    return pl.pallas_call(
        _gru_body,
        grid=grid,
        in_specs=[
            pl.BlockSpec((NB, 1, 1, DIN), lambda n, t: (n, t, 0, 0)),
            pl.BlockSpec((NB, H), lambda n, t: (n, 0)),
            pl.BlockSpec((NB, H), lambda n, t: (n, 0)),
            full((DIN, 3 * H)), full((H, 3 * H)), full((H, 3 * H)), full((H, 3 * H)),
            full((1, 3 * H)), full((1, 3 * H)), full((1, 3 * H)), full((1, 3 * H)),
        ],
        out_specs=pl.BlockSpec((NB, H), lambda n, t: (n, 0)),
        out_shape=jax.ShapeDtypeStruct((N, H), jnp.float32),
        scratch_shapes=[pltpu.VMEM((NB, H), jnp.float32),
                        pltpu.VMEM((NB, H), jnp.float32)],
        compiler_params=pltpu.CompilerParams(
            dimension_semantics=("arbitrary", "arbitrary")),
    )(x_all, h00, h01, wih0t, whh0t, wih1t, whh1t, bih0, bhh0, bih1, bhh1)


def _gcn(x, src, dst, w, b):
    loop = jnp.arange(N)
    s = jnp.concatenate([src, loop])
    d = jnp.concatenate([dst, loop])
    ew = jnp.ones(s.shape[0], dtype=x.dtype)
    deg = jnp.zeros(N, dtype=x.dtype).at[d].add(ew)
    dinv = jnp.where(deg > 0, deg ** -0.5, 0.0)
    norm = dinv[s] * dinv[d]
    h = x @ w
    msg = h[s] * norm[:, None]
    out = jnp.zeros((N, w.shape[1]), dtype=x.dtype).at[d].add(msg)
    return out + b


def kernel(merged_tree_feature, merged_tree_edge_index, indices, emb_table, h0,
           W_ih0, W_hh0, b_ih0, b_hh0, W_ih1, W_hh1, b_ih1, b_hh1,
           W1, b1, W2, b2):
    x_all = emb_table[merged_tree_feature].reshape(N, L, 1, DIN)
    x = _gru(x_all, h0[0], h0[1],
             W_ih0.T, W_hh0.T, W_ih1.T, W_hh1.T,
             b_ih0.reshape(1, -1), b_hh0.reshape(1, -1),
             b_ih1.reshape(1, -1), b_hh1.reshape(1, -1))
    src = merged_tree_edge_index[1]
    dst = merged_tree_edge_index[0]
    x1 = x
    x = _gcn(x, src, dst, W1, b1)
    root_extend = x1[indices]
    x = jnp.concatenate([x, root_extend], axis=1)
    x = jax.nn.elu(x)
    x = _gcn(x, src, dst, W2, b2)
    x = jax.nn.elu(x)
    seg_sum = jax.ops.segment_sum(x, indices, num_segments=B)
    counts = jax.ops.segment_sum(jnp.ones((N,), dtype=x.dtype), indices, num_segments=B)
    return seg_sum / jnp.clip(counts, 1.0)[:, None]


# speedup vs baseline: 5.3180x; 4.2928x over previous
import functools

import jax
import jax.numpy as jnp
from jax import lax
from jax.experimental import pallas as pl
from jax.experimental.pallas import tpu as pltpu
from jax.experimental.pallas import tpu_sc as plsc

N = 10000
L = 20
V = 30000
DIN = 100
H = 128
H1 = 100
H2 = 100
E = 160000
B = 128

NB = 1000
NBLK = N // NB
WPAD = 128
NPAD = 10240
RPT = NPAD // 16
NW = 32
CH = 40
EPAD = NW * CH * 128


def _gru_body(x_ref, h00_ref, h01_ref,
              wih0_ref, whh0_ref, wih1_ref, whh1_ref,
              bih0_ref, bhh0_ref, bih1_ref, bhh1_ref,
              out_ref, ha, hb):
    t = pl.program_id(1)

    @pl.when(t == 0)
    def _init():
        ha[...] = h00_ref[...]
        hb[...] = h01_ref[...]

    def gates(gi, gh, h_prev):
        i_r, i_z, i_n = gi[:, :H], gi[:, H:2 * H], gi[:, 2 * H:]
        h_r, h_z, h_n = gh[:, :H], gh[:, H:2 * H], gh[:, 2 * H:]
        r = jax.nn.sigmoid(i_r + h_r)
        z = jax.nn.sigmoid(i_z + h_z)
        n = jnp.tanh(i_n + r * h_n)
        return (1.0 - z) * n + z * h_prev

    x_t = x_ref[:, 0, 0, :]
    h_a = ha[...]
    h_b = hb[...]
    gi0 = jnp.dot(x_t, wih0_ref[...], preferred_element_type=jnp.float32) + bih0_ref[...]
    gh0 = jnp.dot(h_a, whh0_ref[...], preferred_element_type=jnp.float32) + bhh0_ref[...]
    h_a = gates(gi0, gh0, h_a)
    gi1 = jnp.dot(h_a, wih1_ref[...], preferred_element_type=jnp.float32) + bih1_ref[...]
    gh1 = jnp.dot(h_b, whh1_ref[...], preferred_element_type=jnp.float32) + bhh1_ref[...]
    h_b = gates(gi1, gh1, h_b)
    ha[...] = h_a
    hb[...] = h_b

    @pl.when(t == L - 1)
    def _fin():
        out_ref[...] = h_b


def _gru(x_all, h00, h01, wih0t, whh0t, wih1t, whh1t, bih0, bhh0, bih1, bhh1):
    grid = (NBLK, L)
    full = lambda shape: pl.BlockSpec(shape, lambda n, t: (0,) * len(shape))
    return pl.pallas_call(
        _gru_body,
        grid=grid,
        in_specs=[
            pl.BlockSpec((NB, 1, 1, DIN), lambda n, t: (n, t, 0, 0)),
            pl.BlockSpec((NB, H), lambda n, t: (n, 0)),
            pl.BlockSpec((NB, H), lambda n, t: (n, 0)),
            full((DIN, 3 * H)), full((H, 3 * H)), full((H, 3 * H)), full((H, 3 * H)),
            full((1, 3 * H)), full((1, 3 * H)), full((1, 3 * H)), full((1, 3 * H)),
        ],
        out_specs=pl.BlockSpec((NB, H), lambda n, t: (n, 0)),
        out_shape=jax.ShapeDtypeStruct((N, H), jnp.float32),
        scratch_shapes=[pltpu.VMEM((NB, H), jnp.float32),
                        pltpu.VMEM((NB, H), jnp.float32)],
        compiler_params=pltpu.CompilerParams(
            dimension_semantics=("arbitrary", "arbitrary")),
    )(x_all, h00, h01, wih0t, whh0t, wih1t, whh1t, bih0, bhh0, bih1, bhh1)


def _mesh():
    return plsc.VectorSubcoreMesh(core_axis_name="c", subcore_axis_name="s")


def _sc_deg(dst3, ones16, zeros16):
    @functools.partial(
        pl.kernel,
        out_type=jax.ShapeDtypeStruct((2, NPAD, 16), jnp.float32),
        mesh=_mesh(),
        scratch_types=[
            pltpu.VMEM((CH, 128), jnp.int32),
            pltpu.VMEM((128, 16), jnp.float32),
            pltpu.VMEM_SHARED((NPAD, 16), jnp.float32),
        ],
    )
    def k(dst_hbm, ones_hbm, z_hbm, out_hbm, didx, ones_v, acc):
        c = lax.axis_index("c")
        s = lax.axis_index("s")
        wid = s * 2 + c
        r0 = s * RPT
        pltpu.sync_copy(z_hbm.at[pl.ds(r0, RPT)], acc.at[pl.ds(r0, RPT)])
        pltpu.sync_copy(dst_hbm.at[wid], didx)
        pltpu.sync_copy(ones_hbm, ones_v)
        plsc.subcore_barrier()

        def body(j, carry):
            pltpu.sync_copy(ones_v, acc.at[didx.at[j]], add=True)
            return carry

        lax.fori_loop(0, CH, body, 0)
        plsc.subcore_barrier()
        pltpu.sync_copy(acc.at[pl.ds(r0, RPT)], out_hbm.at[c, pl.ds(r0, RPT)])

    return k(dst3, ones16, zeros16)


def _sc_conv(g, src3, dst3, zeros):
    @functools.partial(
        pl.kernel,
        out_type=jax.ShapeDtypeStruct((2, NPAD, WPAD), jnp.float32),
        mesh=_mesh(),
        scratch_types=[
            pltpu.VMEM((CH, 128), jnp.int32),
            pltpu.VMEM((CH, 128), jnp.int32),
            pltpu.VMEM((128, WPAD), jnp.float32),
            pltpu.VMEM_SHARED((NPAD, WPAD), jnp.float32),
            pltpu.SemaphoreType.DMA,
        ],
    )
    def k(g_hbm, src_hbm, dst_hbm, z_hbm, out_hbm, sidx, didx, rows, acc, sem):
        c = lax.axis_index("c")
        s = lax.axis_index("s")
        wid = s * 2 + c
        r0 = s * RPT
        pltpu.sync_copy(z_hbm.at[pl.ds(r0, RPT)], acc.at[pl.ds(r0, RPT)])
        pltpu.sync_copy(src_hbm.at[wid], sidx)
        pltpu.sync_copy(dst_hbm.at[wid], didx)
        plsc.subcore_barrier()

        def body(j, carry):
            pltpu.async_copy(g_hbm.at[sidx.at[j]], rows, sem).wait()
            pltpu.sync_copy(rows, acc.at[didx.at[j]], add=True)
            return carry

        lax.fori_loop(0, CH, body, 0)
        plsc.subcore_barrier()
        pltpu.sync_copy(acc.at[pl.ds(r0, RPT)], out_hbm.at[c, pl.ds(r0, RPT)])

    return k(g, src3, dst3, zeros)


def _scale_body(x_ref, deg_ref, w_ref, out_ref):
    deg = deg_ref[0, :, :] + deg_ref[1, :, :] + 1.0
    dinv = lax.rsqrt(deg[:, 0:1])
    h = jnp.dot(x_ref[...], w_ref[...], preferred_element_type=jnp.float32)
    out_ref[...] = h * dinv


def _tc_scale(x, deg, w):
    kdim = w.shape[0]
    return pl.pallas_call(
        _scale_body,
        grid=(NBLK,),
        in_specs=[
            pl.BlockSpec((NB, kdim), lambda n: (n, 0)),
            pl.BlockSpec((2, NB, 16), lambda n: (0, n, 0)),
            pl.BlockSpec((kdim, WPAD), lambda n: (0, 0)),
        ],
        out_specs=pl.BlockSpec((NB, WPAD), lambda n: (n, 0)),
        out_shape=jax.ShapeDtypeStruct((NPAD, WPAD), jnp.float32),
    )(x, deg, w)


def _combine1_body(g_ref, acc_ref, deg_ref, xr_ref, idx_ref, b1_ref, w2_ref,
                   out_ref):
    deg = deg_ref[0, :, :] + deg_ref[1, :, :] + 1.0
    dinv = lax.rsqrt(deg[:, 0:1])
    agg = g_ref[...] + acc_ref[0, :, :] + acc_ref[1, :, :]
    c1 = agg * dinv + b1_ref[...]
    idx = idx_ref[0, 0, :]
    onehot = (idx[:, None] == lax.broadcasted_iota(jnp.int32, (NB, B), 1)
              ).astype(jnp.float32)
    root = jnp.dot(onehot, xr_ref[...], preferred_element_type=jnp.float32)
    z = jnp.concatenate([c1, root], axis=1)
    z = jnp.where(z > 0, z, jnp.exp(z) - 1.0)
    g2 = jnp.dot(z, w2_ref[...], preferred_element_type=jnp.float32)
    out_ref[...] = g2 * dinv


def _tc_combine1(g1, acc1, deg, x, idx3, b1p, w2p):
    return pl.pallas_call(
        _combine1_body,
        grid=(NBLK,),
        in_specs=[
            pl.BlockSpec((NB, WPAD), lambda n: (n, 0)),
            pl.BlockSpec((2, NB, WPAD), lambda n: (0, n, 0)),
            pl.BlockSpec((2, NB, 16), lambda n: (0, n, 0)),
            pl.BlockSpec((B, H), lambda n: (0, 0)),
            pl.BlockSpec((1, 1, NB), lambda n: (n, 0, 0)),
            pl.BlockSpec((1, WPAD), lambda n: (0, 0)),
            pl.BlockSpec((2 * WPAD, WPAD), lambda n: (0, 0)),
        ],
        out_specs=pl.BlockSpec((NB, WPAD), lambda n: (n, 0)),
        out_shape=jax.ShapeDtypeStruct((NPAD, WPAD), jnp.float32),
    )(g1, acc1, deg, x, idx3, b1p, w2p)


def _pool_body(g_ref, acc_ref, deg_ref, idx_ref, b2_ref, out_ref,
               seg_acc, cnt_acc):
    n = pl.program_id(0)
    deg = deg_ref[0, :, :] + deg_ref[1, :, :] + 1.0
    dinv = lax.rsqrt(deg[:, 0:1])
    agg = g_ref[...] + acc_ref[0, :, :] + acc_ref[1, :, :]
    y = agg * dinv + b2_ref[...]
    y = jnp.where(y > 0, y, jnp.exp(y) - 1.0)
    idx = idx_ref[0, 0, :]
    onehot = (idx[:, None] == lax.broadcasted_iota(jnp.int32, (NB, B), 1)
              ).astype(jnp.float32)
    dn = (((0,), (0,)), ((), ()))
    seg = lax.dot_general(onehot, y, dn, preferred_element_type=jnp.float32)
    ones = jnp.ones((NB, WPAD), dtype=jnp.float32)
    cnt = lax.dot_general(onehot, ones, dn, preferred_element_type=jnp.float32)

    @pl.when(n == 0)
    def _z():
        seg_acc[...] = jnp.zeros_like(seg_acc)
        cnt_acc[...] = jnp.zeros_like(cnt_acc)

    seg_acc[...] += seg
    cnt_acc[...] += cnt

    @pl.when(n == NBLK - 1)
    def _f():
        out_ref[...] = (seg_acc[:, :H2] /
                        jnp.maximum(cnt_acc[:, :H2], 1.0))


def _tc_pool(g2, acc2, deg, idx3, b2p):
    return pl.pallas_call(
        _pool_body,
        grid=(NBLK,),
        in_specs=[
            pl.BlockSpec((NB, WPAD), lambda n: (n, 0)),
            pl.BlockSpec((2, NB, WPAD), lambda n: (0, n, 0)),
            pl.BlockSpec((2, NB, 16), lambda n: (0, n, 0)),
            pl.BlockSpec((1, 1, NB), lambda n: (n, 0, 0)),
            pl.BlockSpec((1, WPAD), lambda n: (0, 0)),
        ],
        out_specs=pl.BlockSpec((B, H2), lambda n: (0, 0)),
        out_shape=jax.ShapeDtypeStruct((B, H2), jnp.float32),
        scratch_shapes=[pltpu.VMEM((B, WPAD), jnp.float32),
                        pltpu.VMEM((B, WPAD), jnp.float32)],
        compiler_params=pltpu.CompilerParams(
            dimension_semantics=("arbitrary",)),
    )(g2, acc2, deg, idx3, b2p)


def _pad_edges(e):
    pad = jnp.full((EPAD - E,), N, dtype=jnp.int32)
    return jnp.concatenate([e.astype(jnp.int32), pad]).reshape(NW, CH, 128)


def kernel(merged_tree_feature, merged_tree_edge_index, indices, emb_table, h0,
           W_ih0, W_hh0, b_ih0, b_hh0, W_ih1, W_hh1, b_ih1, b_hh1,
           W1, b1, W2, b2):
    f32 = jnp.float32
    x_all = emb_table[merged_tree_feature].reshape(N, L, 1, DIN)
    x = _gru(x_all, h0[0], h0[1],
             W_ih0.T, W_hh0.T, W_ih1.T, W_hh1.T,
             b_ih0.reshape(1, -1), b_hh0.reshape(1, -1),
             b_ih1.reshape(1, -1), b_hh1.reshape(1, -1))

    src3 = _pad_edges(merged_tree_edge_index[1])
    dst3 = _pad_edges(merged_tree_edge_index[0])
    idx3 = indices.astype(jnp.int32).reshape(NBLK, 1, NB)

    ones16 = jnp.ones((128, 16), f32)
    zeros16 = jnp.zeros((NPAD, 16), f32)
    zerosW = jnp.zeros((NPAD, WPAD), f32)

    w1p = jnp.pad(W1, ((0, 0), (0, WPAD - H1)))
    b1p = jnp.pad(b1, (0, WPAD - H1)).reshape(1, WPAD)
    w2p = jnp.zeros((2 * WPAD, WPAD), f32)
    w2p = w2p.at[:H1, :H2].set(W2[:H1])
    w2p = w2p.at[WPAD:WPAD + H, :H2].set(W2[H1:])
    b2p = jnp.pad(b2, (0, WPAD - H2)).reshape(1, WPAD)

    deg = _sc_deg(dst3, ones16, zeros16)
    g1 = _tc_scale(x, deg, w1p)
    acc1 = _sc_conv(g1, src3, dst3, zerosW)
    g2 = _tc_combine1(g1, acc1, deg, x, idx3, b1p, w2p)
    acc2 = _sc_conv(g2, src3, dst3, zerosW)
    return _tc_pool(g2, acc2, deg, idx3, b2p)


# speedup vs baseline: 7.3731x; 1.3864x over previous
import functools

import jax
import jax.numpy as jnp
from jax import lax
from jax.experimental import pallas as pl
from jax.experimental.pallas import tpu as pltpu
from jax.experimental.pallas import tpu_sc as plsc

N = 10000
L = 20
V = 30000
DIN = 100
H = 128
H1 = 100
H2 = 100
E = 160000
B = 128

NB = 1000
NBLK = N // NB
WPAD = 128
NPAD = 10240
RPT = NPAD // 16
NW = 32
CH = 80
ECC = 64
EPAD = NW * CH * ECC


def _gru_body(x_ref, h00_ref, h01_ref,
              wih0_ref, whh0_ref, wih1_ref, whh1_ref,
              bih0_ref, bhh0_ref, bih1_ref, bhh1_ref,
              out_ref, ha, hb):
    t = pl.program_id(1)

    @pl.when(t == 0)
    def _init():
        ha[...] = h00_ref[...]
        hb[...] = h01_ref[...]

    def gates(gi, gh, h_prev):
        i_r, i_z, i_n = gi[:, :H], gi[:, H:2 * H], gi[:, 2 * H:]
        h_r, h_z, h_n = gh[:, :H], gh[:, H:2 * H], gh[:, 2 * H:]
        r = jax.nn.sigmoid(i_r + h_r)
        z = jax.nn.sigmoid(i_z + h_z)
        n = jnp.tanh(i_n + r * h_n)
        return (1.0 - z) * n + z * h_prev

    x_t = x_ref[...]
    h_a = ha[...]
    h_b = hb[...]
    gi0 = jnp.dot(x_t, wih0_ref[...], preferred_element_type=jnp.float32) + bih0_ref[...]
    gh0 = jnp.dot(h_a, whh0_ref[...], preferred_element_type=jnp.float32) + bhh0_ref[...]
    h_a = gates(gi0, gh0, h_a)
    gi1 = jnp.dot(h_a, wih1_ref[...], preferred_element_type=jnp.float32) + bih1_ref[...]
    gh1 = jnp.dot(h_b, whh1_ref[...], preferred_element_type=jnp.float32) + bhh1_ref[...]
    h_b = gates(gi1, gh1, h_b)
    ha[...] = h_a
    hb[...] = h_b

    @pl.when(t == L - 1)
    def _fin():
        out_ref[...] = h_b


def _gru(x_all, h00, h01, wih0t, whh0t, wih1t, whh1t, bih0, bhh0, bih1, bhh1):
    grid = (NBLK, L)
    full = lambda shape: pl.BlockSpec(shape, lambda n, t: (0,) * len(shape))
    return pl.pallas_call(
        _gru_body,
        grid=grid,
        in_specs=[
            pl.BlockSpec((NB, WPAD), lambda n, t: (t * NBLK + n, 0)),
            pl.BlockSpec((NB, H), lambda n, t: (n, 0)),
            pl.BlockSpec((NB, H), lambda n, t: (n, 0)),
            full((WPAD, 3 * H)), full((H, 3 * H)), full((H, 3 * H)), full((H, 3 * H)),
            full((1, 3 * H)), full((1, 3 * H)), full((1, 3 * H)), full((1, 3 * H)),
        ],
        out_specs=pl.BlockSpec((NB, H), lambda n, t: (n, 0)),
        out_shape=jax.ShapeDtypeStruct((N, H), jnp.float32),
        scratch_shapes=[pltpu.VMEM((NB, H), jnp.float32),
                        pltpu.VMEM((NB, H), jnp.float32)],
        compiler_params=pltpu.CompilerParams(
            dimension_semantics=("arbitrary", "arbitrary")),
    )(x_all, h00, h01, wih0t, whh0t, wih1t, whh1t, bih0, bhh0, bih1, bhh1)


def _mesh():
    return plsc.VectorSubcoreMesh(core_axis_name="c", subcore_axis_name="s")


def _sc_deg(dst3, ones16, zeros16):
    @functools.partial(
        pl.kernel,
        out_type=jax.ShapeDtypeStruct((2, NPAD, 16), jnp.float32),
        mesh=_mesh(),
        scratch_types=[
            pltpu.VMEM((CH, ECC), jnp.int32),
            pltpu.VMEM((ECC, 16), jnp.float32),
            pltpu.VMEM_SHARED((NPAD, 16), jnp.float32),
        ],
    )
    def k(dst_hbm, ones_hbm, z_hbm, out_hbm, didx, ones_v, acc):
        c = lax.axis_index("c")
        s = lax.axis_index("s")
        wid = s * 2 + c
        r0 = s * RPT
        pltpu.sync_copy(z_hbm.at[pl.ds(r0, RPT)], acc.at[pl.ds(r0, RPT)])
        pltpu.sync_copy(dst_hbm.at[wid], didx)
        pltpu.sync_copy(ones_hbm, ones_v)
        plsc.subcore_barrier()

        def body(j, carry):
            pltpu.sync_copy(ones_v, acc.at[didx.at[j]], add=True)
            return carry

        lax.fori_loop(0, CH, body, 0)
        plsc.subcore_barrier()
        pltpu.sync_copy(acc.at[pl.ds(r0, RPT)], out_hbm.at[c, pl.ds(r0, RPT)])

    return k(dst3, ones16, zeros16)


def _sc_conv(g, src3, dst3, zeros):
    NBUF = 2

    @functools.partial(
        pl.kernel,
        out_type=jax.ShapeDtypeStruct((2, NPAD, WPAD), jnp.float32),
        mesh=_mesh(),
        scratch_types=[
            pltpu.VMEM((CH, ECC), jnp.int32),
            pltpu.VMEM((CH, ECC), jnp.int32),
            pltpu.VMEM((NBUF, ECC, WPAD), jnp.float32),
            pltpu.VMEM_SHARED((NPAD, WPAD), jnp.float32),
            pltpu.SemaphoreType.DMA,
        ],
    )
    def k(g_hbm, src_hbm, dst_hbm, z_hbm, out_hbm, sidx, didx, rows, acc, sem):
        c = lax.axis_index("c")
        s = lax.axis_index("s")
        wid = s * 2 + c
        r0 = s * RPT
        pltpu.sync_copy(z_hbm.at[pl.ds(r0, RPT)], acc.at[pl.ds(r0, RPT)])
        pltpu.sync_copy(src_hbm.at[wid], sidx)
        pltpu.sync_copy(dst_hbm.at[wid], didx)
        plsc.subcore_barrier()
        for b in range(NBUF):
            pltpu.async_copy(g_hbm.at[sidx.at[b]], rows.at[b], sem)

        def grp(i, carry):
            j0 = i * NBUF
            for b in range(NBUF):
                j = j0 + b
                pltpu.make_async_copy(g_hbm.at[sidx.at[j]], rows.at[b], sem).wait()
                pltpu.sync_copy(rows.at[b], acc.at[didx.at[j]], add=True)

                @pl.when(j + NBUF < CH)
                def _():
                    pltpu.async_copy(g_hbm.at[sidx.at[j + NBUF]], rows.at[b], sem)
            return carry

        lax.fori_loop(0, CH // NBUF, grp, 0)
        plsc.subcore_barrier()
        pltpu.sync_copy(acc.at[pl.ds(r0, RPT)], out_hbm.at[c, pl.ds(r0, RPT)])

    return k(g, src3, dst3, zeros)


CHE = 98
ECH = 64
XROWS = N * L + NB


def _sc_emb(emb_table, feat3, ow3):
    NBUF = 7

    @functools.partial(
        pl.kernel,
        out_type=jax.ShapeDtypeStruct((XROWS, WPAD), jnp.float32),
        mesh=_mesh(),
        scratch_types=[
            pltpu.VMEM((CHE, ECH), jnp.int32),
            pltpu.VMEM((CHE, ECH), jnp.int32),
            pltpu.VMEM((NBUF, ECH, WPAD), jnp.float32),
            pltpu.SemaphoreType.DMA,
        ],
    )
    def k(tab_hbm, feat_hbm, ow_hbm, out_hbm, fidx, ow, rows, sem):
        c = lax.axis_index("c")
        s = lax.axis_index("s")
        wid = s * 2 + c
        pltpu.sync_copy(feat_hbm.at[wid], fidx)
        pltpu.sync_copy(ow_hbm.at[wid], ow)
        for b in range(NBUF):
            pltpu.async_copy(tab_hbm.at[fidx.at[b]], rows.at[b], sem)

        def grp(i, carry):
            j0 = i * NBUF
            for b in range(NBUF):
                j = j0 + b
                pltpu.make_async_copy(tab_hbm.at[fidx.at[j]], rows.at[b], sem).wait()
                pltpu.sync_copy(rows.at[b], out_hbm.at[ow.at[j]])

                @pl.when(j + NBUF < CHE)
                def _():
                    pltpu.async_copy(tab_hbm.at[fidx.at[j + NBUF]], rows.at[b], sem)
            return carry

        lax.fori_loop(0, CHE // NBUF, grp, 0)

    return k(emb_table, feat3, ow3)


def _scale_body(x_ref, deg_ref, w_ref, out_ref):
    deg = deg_ref[0, :, :] + deg_ref[1, :, :] + 1.0
    dinv = lax.rsqrt(deg[:, 0:1])
    h = jnp.dot(x_ref[...], w_ref[...], preferred_element_type=jnp.float32)
    out_ref[...] = h * dinv


def _tc_scale(x, deg, w):
    kdim = w.shape[0]
    return pl.pallas_call(
        _scale_body,
        grid=(NBLK,),
        in_specs=[
            pl.BlockSpec((NB, kdim), lambda n: (n, 0)),
            pl.BlockSpec((2, NB, 16), lambda n: (0, n, 0)),
            pl.BlockSpec((kdim, WPAD), lambda n: (0, 0)),
        ],
        out_specs=pl.BlockSpec((NB, WPAD), lambda n: (n, 0)),
        out_shape=jax.ShapeDtypeStruct((NPAD, WPAD), jnp.float32),
    )(x, deg, w)


def _combine1_body(g_ref, acc_ref, deg_ref, xr_ref, idx_ref, b1_ref, w2_ref,
                   out_ref):
    deg = deg_ref[0, :, :] + deg_ref[1, :, :] + 1.0
    dinv = lax.rsqrt(deg[:, 0:1])
    agg = g_ref[...] + acc_ref[0, :, :] + acc_ref[1, :, :]
    c1 = agg * dinv + b1_ref[...]
    idx = idx_ref[0, 0, :]
    onehot = (idx[:, None] == lax.broadcasted_iota(jnp.int32, (NB, B), 1)
              ).astype(jnp.float32)
    root = jnp.dot(onehot, xr_ref[...], preferred_element_type=jnp.float32)
    z = jnp.concatenate([c1, root], axis=1)
    z = jnp.where(z > 0, z, jnp.exp(z) - 1.0)
    g2 = jnp.dot(z, w2_ref[...], preferred_element_type=jnp.float32)
    out_ref[...] = g2 * dinv


def _tc_combine1(g1, acc1, deg, x, idx3, b1p, w2p):
    return pl.pallas_call(
        _combine1_body,
        grid=(NBLK,),
        in_specs=[
            pl.BlockSpec((NB, WPAD), lambda n: (n, 0)),
            pl.BlockSpec((2, NB, WPAD), lambda n: (0, n, 0)),
            pl.BlockSpec((2, NB, 16), lambda n: (0, n, 0)),
            pl.BlockSpec((B, H), lambda n: (0, 0)),
            pl.BlockSpec((1, 1, NB), lambda n: (n, 0, 0)),
            pl.BlockSpec((1, WPAD), lambda n: (0, 0)),
            pl.BlockSpec((2 * WPAD, WPAD), lambda n: (0, 0)),
        ],
        out_specs=pl.BlockSpec((NB, WPAD), lambda n: (n, 0)),
        out_shape=jax.ShapeDtypeStruct((NPAD, WPAD), jnp.float32),
    )(g1, acc1, deg, x, idx3, b1p, w2p)


def _pool_body(g_ref, acc_ref, deg_ref, idx_ref, b2_ref, out_ref,
               seg_acc, cnt_acc):
    n = pl.program_id(0)
    deg = deg_ref[0, :, :] + deg_ref[1, :, :] + 1.0
    dinv = lax.rsqrt(deg[:, 0:1])
    agg = g_ref[...] + acc_ref[0, :, :] + acc_ref[1, :, :]
    y = agg * dinv + b2_ref[...]
    y = jnp.where(y > 0, y, jnp.exp(y) - 1.0)
    idx = idx_ref[0, 0, :]
    onehot = (idx[:, None] == lax.broadcasted_iota(jnp.int32, (NB, B), 1)
              ).astype(jnp.float32)
    dn = (((0,), (0,)), ((), ()))
    seg = lax.dot_general(onehot, y, dn, preferred_element_type=jnp.float32)
    ones = jnp.ones((NB, WPAD), dtype=jnp.float32)
    cnt = lax.dot_general(onehot, ones, dn, preferred_element_type=jnp.float32)

    @pl.when(n == 0)
    def _z():
        seg_acc[...] = jnp.zeros_like(seg_acc)
        cnt_acc[...] = jnp.zeros_like(cnt_acc)

    seg_acc[...] += seg
    cnt_acc[...] += cnt

    @pl.when(n == NBLK - 1)
    def _f():
        out_ref[...] = (seg_acc[:, :H2] /
                        jnp.maximum(cnt_acc[:, :H2], 1.0))


def _tc_pool(g2, acc2, deg, idx3, b2p):
    return pl.pallas_call(
        _pool_body,
        grid=(NBLK,),
        in_specs=[
            pl.BlockSpec((NB, WPAD), lambda n: (n, 0)),
            pl.BlockSpec((2, NB, WPAD), lambda n: (0, n, 0)),
            pl.BlockSpec((2, NB, 16), lambda n: (0, n, 0)),
            pl.BlockSpec((1, 1, NB), lambda n: (n, 0, 0)),
            pl.BlockSpec((1, WPAD), lambda n: (0, 0)),
        ],
        out_specs=pl.BlockSpec((B, H2), lambda n: (0, 0)),
        out_shape=jax.ShapeDtypeStruct((B, H2), jnp.float32),
        scratch_shapes=[pltpu.VMEM((B, WPAD), jnp.float32),
                        pltpu.VMEM((B, WPAD), jnp.float32)],
        compiler_params=pltpu.CompilerParams(
            dimension_semantics=("arbitrary",)),
    )(g2, acc2, deg, idx3, b2p)


def _pad_edges(e):
    pad = jnp.full((EPAD - E,), N, dtype=jnp.int32)
    return jnp.concatenate([e.astype(jnp.int32), pad]).reshape(NW, CH, ECC)


def kernel(merged_tree_feature, merged_tree_edge_index, indices, emb_table, h0,
           W_ih0, W_hh0, b_ih0, b_hh0, W_ih1, W_hh1, b_ih1, b_hh1,
           W1, b1, W2, b2):
    f32 = jnp.float32
    i32 = jnp.int32
    fpad = NW * CHE * ECH - N * L
    feat3 = jnp.concatenate(
        [merged_tree_feature.astype(i32).reshape(-1),
         jnp.zeros((fpad,), i32)]).reshape(NW, CHE, ECH)
    p = jnp.arange(NW * CHE * ECH, dtype=i32)
    ow3 = jnp.where(p < N * L, (p % L) * N + p // L, N * L).reshape(NW, CHE, ECH)
    x_all = _sc_emb(jnp.pad(emb_table, ((0, 0), (0, WPAD - DIN))), feat3, ow3)
    x = _gru(x_all, h0[0], h0[1],
             jnp.pad(W_ih0.T, ((0, WPAD - DIN), (0, 0))),
             W_hh0.T, W_ih1.T, W_hh1.T,
             b_ih0.reshape(1, -1), b_hh0.reshape(1, -1),
             b_ih1.reshape(1, -1), b_hh1.reshape(1, -1))

    src3 = _pad_edges(merged_tree_edge_index[1])
    dst3 = _pad_edges(merged_tree_edge_index[0])
    idx3 = indices.astype(jnp.int32).reshape(NBLK, 1, NB)

    ones16 = jnp.ones((ECC, 16), f32)
    zeros16 = jnp.zeros((NPAD, 16), f32)
    zerosW = jnp.zeros((NPAD, WPAD), f32)

    w1p = jnp.pad(W1, ((0, 0), (0, WPAD - H1)))
    b1p = jnp.pad(b1, (0, WPAD - H1)).reshape(1, WPAD)
    w2p = jnp.zeros((2 * WPAD, WPAD), f32)
    w2p = w2p.at[:H1, :H2].set(W2[:H1])
    w2p = w2p.at[WPAD:WPAD + H, :H2].set(W2[H1:])
    b2p = jnp.pad(b2, (0, WPAD - H2)).reshape(1, WPAD)

    deg = _sc_deg(dst3, ones16, zeros16)
    g1 = _tc_scale(x, deg, w1p)
    acc1 = _sc_conv(g1, src3, dst3, zerosW)
    g2 = _tc_combine1(g1, acc1, deg, x, idx3, b1p, w2p)
    acc2 = _sc_conv(g2, src3, dst3, zerosW)
    return _tc_pool(g2, acc2, deg, idx3, b2p)
